# baseline (speedup 1.0000x reference)
import functools

import jax
import jax.numpy as jnp
from jax.experimental import pallas as pl
from jax.experimental.pallas import tpu as pltpu


def _round_up(v, m):
    return (v + m - 1) // m * m


def _fused_kernel(x_hbm, w_ref, b_ref, a_hbm, o_hbm,
                  x_vmem, xw_ref, a_buf, o_buf,
                  x_sem, in_sem, out_sem, *, tm, steps):
    hm = tm // 2

    def a_copy_half(slot, step, half):
        return pltpu.make_async_copy(
            a_hbm.at[pl.ds(step * tm + half * hm, hm)],
            a_buf.at[slot].at[pl.ds(half * hm, hm)],
            in_sem.at[slot, half])

    class _APair:
        def __init__(self, slot, step):
            self.halves = (a_copy_half(slot, step, 0), a_copy_half(slot, step, 1))

        def start(self):
            for c in self.halves:
                c.start()

        def wait(self):
            for c in self.halves:
                c.wait()

    def a_copy(slot, step):
        return _APair(slot, step)

    def o_copy(slot, step):
        return pltpu.make_async_copy(
            o_buf.at[slot], o_hbm.at[pl.ds(step * tm, tm)], out_sem.at[slot])

    x_copy = pltpu.make_async_copy(x_hbm, x_vmem, x_sem)
    x_copy.start()
    a_copy(0, 0).start()
    if steps > 1:
        a_copy(1, 1).start()

    x_copy.wait()
    xw_ref[...] = jnp.dot(
        x_vmem[...].astype(jnp.bfloat16),
        w_ref[...].astype(jnp.bfloat16),
        preferred_element_type=jnp.float32,
    ).astype(jnp.bfloat16)

    for step in range(steps):
        cur = step % 3
        a_copy(cur, step).wait()
        if step + 2 < steps:
            a_copy((step + 2) % 3, step + 2).start()
        if step >= 2:
            o_copy(step % 2, step - 2).wait()
        acc = jnp.dot(
            a_buf[cur].astype(jnp.bfloat16), xw_ref[...],
            preferred_element_type=jnp.float32,
        )
        o_buf[step % 2] = jnp.maximum(acc + b_ref[...], 0.0)
        o_copy(step % 2, step).start()

    for step in range(max(steps - 2, 0), steps):
        o_copy(step % 2, step).wait()


def _gcn_fused(a, x, w, b2d, *, tm):
    n, h = x.shape
    steps = n // tm
    return pl.pallas_call(
        functools.partial(_fused_kernel, tm=tm, steps=steps),
        out_shape=jax.ShapeDtypeStruct((n, h), jnp.float32),
        in_specs=[
            pl.BlockSpec(memory_space=pl.ANY),
            pl.BlockSpec(memory_space=pltpu.MemorySpace.VMEM),
            pl.BlockSpec(memory_space=pltpu.MemorySpace.VMEM),
            pl.BlockSpec(memory_space=pl.ANY),
        ],
        out_specs=pl.BlockSpec(memory_space=pl.ANY),
        scratch_shapes=[
            pltpu.VMEM((n, h), jnp.float32),
            pltpu.VMEM((n, h), jnp.bfloat16),
            pltpu.VMEM((3, tm, n), jnp.float32),
            pltpu.VMEM((2, tm, h), jnp.float32),
            pltpu.SemaphoreType.DMA,
            pltpu.SemaphoreType.DMA((3, 2)),
            pltpu.SemaphoreType.DMA((2,)),
        ],
        compiler_params=pltpu.CompilerParams(
            vmem_limit_bytes=100 * 1024 * 1024,
        ),
    )(x, w, b2d, a)


def kernel(a_norm, x, w, b):
    n, h = x.shape
    n_pad = _round_up(n, 512)
    h_pad = _round_up(h, 128)
    if n_pad != n or h_pad != h:
        a_norm = jnp.pad(a_norm, ((0, n_pad - n), (0, n_pad - n)))
        x = jnp.pad(x, ((0, n_pad - n), (0, h_pad - h)))
        w = jnp.pad(w, ((0, h_pad - h), (0, h_pad - h)))
        b = jnp.pad(b, (0, h_pad - h))
    b2d = b.reshape(1, h_pad).astype(jnp.float32)

    out = _gcn_fused(a_norm, x, w, b2d, tm=512)
    return out[:n, :h]


# speedup vs baseline: 1.0057x; 1.0057x over previous
import functools

import jax
import jax.numpy as jnp
from jax.experimental import pallas as pl
from jax.experimental.pallas import tpu as pltpu


def _round_up(v, m):
    return (v + m - 1) // m * m


def _schedule(n):
    if n <= 512:
        return [n]
    sizes = [256]
    rem = n - 256 - 384
    while rem >= 512:
        sizes.append(512)
        rem -= 512
    if rem:
        sizes.append(rem)
    sizes.extend([256, 128])
    return sizes


def _fused_kernel(x_hbm, w_ref, b_ref, a_hbm, o_hbm,
                  x_vmem, xw_ref, a_buf, o_buf,
                  x_sem, in_sem, out_sem, *, sizes, offs):
    steps = len(sizes)

    def a_copy(slot, step):
        return pltpu.make_async_copy(
            a_hbm.at[pl.ds(offs[step], sizes[step])],
            a_buf.at[slot].at[pl.ds(0, sizes[step])],
            in_sem.at[slot])

    def o_copy(slot, step):
        return pltpu.make_async_copy(
            o_buf.at[slot].at[pl.ds(0, sizes[step])],
            o_hbm.at[pl.ds(offs[step], sizes[step])],
            out_sem.at[slot])

    x_copy = pltpu.make_async_copy(x_hbm, x_vmem, x_sem)
    x_copy.start()
    a_copy(0, 0).start()
    if steps > 1:
        a_copy(1, 1).start()

    x_copy.wait()
    xw_ref[...] = jnp.dot(
        x_vmem[...].astype(jnp.bfloat16),
        w_ref[...].astype(jnp.bfloat16),
        preferred_element_type=jnp.float32,
    ).astype(jnp.bfloat16)

    for step in range(steps):
        cur = step % 3
        a_copy(cur, step).wait()
        if step + 2 < steps:
            a_copy((step + 2) % 3, step + 2).start()
        if step >= 2:
            o_copy(step % 2, step - 2).wait()
        acc = jnp.dot(
            a_buf[cur, pl.ds(0, sizes[step])].astype(jnp.bfloat16), xw_ref[...],
            preferred_element_type=jnp.float32,
        )
        o_buf[step % 2, pl.ds(0, sizes[step])] = jnp.maximum(acc + b_ref[...], 0.0)
        o_copy(step % 2, step).start()

    for step in range(max(steps - 2, 0), steps):
        o_copy(step % 2, step).wait()


def _gcn_fused(a, x, w, b2d):
    n, h = x.shape
    sizes = _schedule(n)
    offs = [0]
    for s in sizes[:-1]:
        offs.append(offs[-1] + s)
    tm = max(sizes)
    return pl.pallas_call(
        functools.partial(_fused_kernel, sizes=tuple(sizes), offs=tuple(offs)),
        out_shape=jax.ShapeDtypeStruct((n, h), jnp.float32),
        in_specs=[
            pl.BlockSpec(memory_space=pl.ANY),
            pl.BlockSpec(memory_space=pltpu.MemorySpace.VMEM),
            pl.BlockSpec(memory_space=pltpu.MemorySpace.VMEM),
            pl.BlockSpec(memory_space=pl.ANY),
        ],
        out_specs=pl.BlockSpec(memory_space=pl.ANY),
        scratch_shapes=[
            pltpu.VMEM((n, h), jnp.float32),
            pltpu.VMEM((n, h), jnp.bfloat16),
            pltpu.VMEM((3, tm, n), jnp.float32),
            pltpu.VMEM((2, tm, h), jnp.float32),
            pltpu.SemaphoreType.DMA,
            pltpu.SemaphoreType.DMA((3,)),
            pltpu.SemaphoreType.DMA((2,)),
        ],
        compiler_params=pltpu.CompilerParams(
            vmem_limit_bytes=100 * 1024 * 1024,
        ),
    )(x, w, b2d, a)


def kernel(a_norm, x, w, b):
    n, h = x.shape
    n_pad = _round_up(n, 128)
    h_pad = _round_up(h, 128)
    if n_pad != n or h_pad != h:
        a_norm = jnp.pad(a_norm, ((0, n_pad - n), (0, n_pad - n)))
        x = jnp.pad(x, ((0, n_pad - n), (0, h_pad - h)))
        w = jnp.pad(w, ((0, h_pad - h), (0, h_pad - h)))
        b = jnp.pad(b, (0, h_pad - h))
    b2d = b.reshape(1, h_pad).astype(jnp.float32)

    out = _gcn_fused(a_norm, x, w, b2d)
    return out[:n, :h]


# speedup vs baseline: 1.0306x; 1.0248x over previous
import jax
import jax.numpy as jnp
from jax.experimental import pallas as pl
from jax.experimental.pallas import tpu as pltpu


def _round_up(v, m):
    return (v + m - 1) // m * m


def _fused_kernel(x_ref, w_ref, b_ref, a_ref, o_ref, xw_ref):
    i = pl.program_id(0)

    @pl.when(i == 0)
    def _():
        xw_ref[...] = jnp.dot(
            x_ref[...].astype(jnp.bfloat16),
            w_ref[...].astype(jnp.bfloat16),
            preferred_element_type=jnp.float32,
        ).astype(jnp.bfloat16)

    @pl.when(i > 0)
    def _():
        a_bf = a_ref[...].astype(jnp.bfloat16)
        acc = jnp.dot(a_bf, xw_ref[...], preferred_element_type=jnp.float32)
        o_ref[...] = jnp.maximum(acc + b_ref[...], 0.0)


def _gcn_fused(a, x, w, b2d, *, tm):
    n, h = x.shape
    steps = n // tm

    def a_idx(i):
        return (jnp.maximum(i - 1, 0), 0)

    def o_idx(i):
        return (jnp.maximum(i - 1, 0), 0)

    return pl.pallas_call(
        _fused_kernel,
        out_shape=jax.ShapeDtypeStruct((n, h), jnp.float32),
        grid=(steps + 1,),
        in_specs=[
            pl.BlockSpec((n, h), lambda i: (0, 0)),
            pl.BlockSpec((h, h), lambda i: (0, 0)),
            pl.BlockSpec((1, h), lambda i: (0, 0)),
            pl.BlockSpec((tm, n), a_idx),
        ],
        out_specs=pl.BlockSpec((tm, h), o_idx),
        scratch_shapes=[pltpu.VMEM((n, h), jnp.bfloat16)],
        compiler_params=pltpu.CompilerParams(
            dimension_semantics=("arbitrary",),
            vmem_limit_bytes=100 * 1024 * 1024,
        ),
    )(x, w, b2d, a)


def kernel(a_norm, x, w, b):
    n, h = x.shape
    n_pad = _round_up(n, 512)
    h_pad = _round_up(h, 128)
    if n_pad != n or h_pad != h:
        a_norm = jnp.pad(a_norm, ((0, n_pad - n), (0, n_pad - n)))
        x = jnp.pad(x, ((0, n_pad - n), (0, h_pad - h)))
        w = jnp.pad(w, ((0, h_pad - h), (0, h_pad - h)))
        b = jnp.pad(b, (0, h_pad - h))
    b2d = b.reshape(1, h_pad).astype(jnp.float32)

    out = _gcn_fused(a_norm, x, w, b2d, tm=512)
    return out[:n, :h]
